# baseline (speedup 1.0000x reference)
import functools

import jax
import jax.numpy as jnp
from jax import lax
from jax.experimental import pallas as pl
from jax.experimental.pallas import tpu as pltpu
from jax.experimental.pallas import tpu_sc as plsc

D_MODEL = 64
SCALE = 8.0
LANES = 16

NW = 32
G = 128
CHUNK = 512
GPC = CHUNK // G


def _emb_kernel(idx_hbm, lut_hbm, out_hbm, idx_v, rows_v, sem, *, b_per_w):
    wid = lax.axis_index("s") * 2 + lax.axis_index("c")
    chunk_rows0 = wid * (b_per_w // G)
    n_chunks = b_per_w // CHUNK

    def chunk_body(ci, _):
        row0 = chunk_rows0 + ci * GPC
        pltpu.sync_copy(idx_hbm.at[pl.ds(row0, GPC)], idx_v)
        cps = [
            pltpu.async_copy(
                lut_hbm.at[idx_v.at[j]], rows_v.at[pl.ds(j * G, G)], sem
            )
            for j in range(GPC)
        ]
        for cp in cps:
            cp.wait()

        def row_body(r, _):
            for j in range(D_MODEL // LANES):
                sl = pl.ds(j * LANES, LANES)
                rows_v[r, sl] = rows_v[r, sl] * SCALE
            return 0

        lax.fori_loop(0, CHUNK, row_body, 0)
        pltpu.sync_copy(rows_v, out_hbm.at[pl.ds(row0 * G, CHUNK)])
        return 0

    lax.fori_loop(0, n_chunks, chunk_body, 0)


def kernel(x, lut):
    s0, s1 = x.shape
    B = s0 * s1
    b_per_w = B // NW
    idx = x.reshape(B // G, G).astype(jnp.int32)

    mesh = plsc.VectorSubcoreMesh(core_axis_name="c", subcore_axis_name="s")
    k = pl.kernel(
        functools.partial(_emb_kernel, b_per_w=b_per_w),
        mesh=mesh,
        out_type=jax.ShapeDtypeStruct((B, D_MODEL), jnp.float32),
        scratch_types=[
            pltpu.VMEM((GPC, G), jnp.int32),
            pltpu.VMEM((CHUNK, D_MODEL), jnp.float32),
            pltpu.SemaphoreType.DMA,
        ],
        compiler_params=pltpu.CompilerParams(use_tc_tiling_on_sc=False),
    )
    out = k(idx, lut)
    return out.reshape(s0, s1, D_MODEL)


# speedup vs baseline: 1.1353x; 1.1353x over previous
import functools

import jax
import jax.numpy as jnp
from jax import lax
from jax.experimental import pallas as pl
from jax.experimental.pallas import tpu as pltpu
from jax.experimental.pallas import tpu_sc as plsc

D_MODEL = 64
SCALE = 8.0
LANES = 16

NW = 32
G = 128
CHUNK = 256
GPC = CHUNK // G


def _emb_kernel(idx_hbm, lut_hbm, out_hbm, idx_all, in0, in1, out0, out1,
                gsem0, gsem1, ssem0, ssem1, *, b_per_w):
    wid = lax.axis_index("s") * 2 + lax.axis_index("c")
    n_groups = b_per_w // G
    n_chunks = b_per_w // CHUNK
    pairs = n_chunks // 2
    tile_g0 = wid * n_groups
    tile_row0 = wid * b_per_w

    bufs = ((in0, out0, gsem0, ssem0), (in1, out1, gsem1, ssem1))

    pltpu.sync_copy(idx_hbm.at[pl.ds(tile_g0, n_groups)], idx_all)

    def fire_gather(c, buf):
        inb, _, gsem, _ = buf
        for j in range(GPC):
            pltpu.async_copy(
                lut_hbm.at[idx_all.at[c * GPC + j]],
                inb.at[pl.ds(j * G, G)],
                gsem,
            )

    def drain_gather(buf):
        inb, _, gsem, _ = buf
        pltpu.make_async_copy(lut_hbm.at[pl.ds(0, CHUNK)], inb, gsem).wait()

    def fire_store(c, buf):
        _, outb, _, ssem = buf
        pltpu.async_copy(outb, out_hbm.at[pl.ds(tile_row0 + c * CHUNK, CHUNK)], ssem)

    def drain_store(buf):
        _, outb, _, ssem = buf
        pltpu.make_async_copy(outb, out_hbm.at[pl.ds(0, CHUNK)], ssem).wait()

    def scale_chunk(buf):
        inb, outb, _, _ = buf

        @plsc.parallel_loop(0, CHUNK, step=1, unroll=8)
        def _(r):
            for j in range(D_MODEL // LANES):
                sl = pl.ds(j * LANES, LANES)
                outb[r, sl] = inb[r, sl] * SCALE

    for b in (0, 1):
        fire_gather(b, bufs[b])
    for b in (0, 1):
        drain_gather(bufs[b])
        scale_chunk(bufs[b])
        fire_store(b, bufs[b])
        fire_gather(b + 2, bufs[b])

    def body(j, _):
        for b in (0, 1):
            c = 2 * j + b
            buf = bufs[b]
            drain_gather(buf)
            drain_store(buf)
            scale_chunk(buf)
            fire_store(c, buf)
            fire_gather(c + 2, buf)
        return 0

    lax.fori_loop(1, pairs - 1, body, 0)

    for b in (0, 1):
        c = 2 * (pairs - 1) + b
        drain_gather(bufs[b])
        drain_store(bufs[b])
        scale_chunk(bufs[b])
        fire_store(c, bufs[b])
    for b in (0, 1):
        drain_store(bufs[b])


def kernel(x, lut):
    s0, s1 = x.shape
    B = s0 * s1
    b_per_w = B // NW
    idx = x.reshape(B // G, G).astype(jnp.int32)

    mesh = plsc.VectorSubcoreMesh(core_axis_name="c", subcore_axis_name="s")
    k = pl.kernel(
        functools.partial(_emb_kernel, b_per_w=b_per_w),
        mesh=mesh,
        out_type=jax.ShapeDtypeStruct((B, D_MODEL), jnp.float32),
        scratch_types=[
            pltpu.VMEM((b_per_w // G, G), jnp.int32),
            pltpu.VMEM((CHUNK, D_MODEL), jnp.float32),
            pltpu.VMEM((CHUNK, D_MODEL), jnp.float32),
            pltpu.VMEM((CHUNK, D_MODEL), jnp.float32),
            pltpu.VMEM((CHUNK, D_MODEL), jnp.float32),
            pltpu.SemaphoreType.DMA,
            pltpu.SemaphoreType.DMA,
            pltpu.SemaphoreType.DMA,
            pltpu.SemaphoreType.DMA,
        ],
        compiler_params=pltpu.CompilerParams(use_tc_tiling_on_sc=False),
    )
    out = k(idx, lut)
    return out.reshape(s0, s1, D_MODEL)
